# Initial kernel scaffold; baseline (speedup 1.0000x reference)
import jax
import jax.numpy as jnp
from jax.experimental import pallas as pl


def kernel(x, edge_index, missing_mask, Wf, bf, Emiss, Etype, Wp, bp, Wq, bq, Wk, bk, Wv, bv, Ws, bs, ln_g, ln_b, P1W, P1b, P2W, P2b, P3W, P3b, U1W, U1b, U2W, U2b, G1W, G1b, G2W, G2b):
    raise NotImplementedError("write your pallas kernel here")


# speedup vs baseline: 23.9338x; 23.9338x over previous
import functools

import jax
import jax.numpy as jnp
from jax import lax
from jax.experimental import pallas as pl
from jax.experimental.pallas import tpu as pltpu
from jax.experimental.pallas import tpu_sc as plsc

N = 10000
D = 32
HID = 128
HEADS = 8
HC = 16
E = 640000
NL = 4
Q = HID // 4

NSC = 2
NSUB = 16
NW = NSC * NSUB
EPW = E // NW
CH = 80
NCHUNK = EPW // CH
GRP = CH // 16

_f32 = jnp.float32


def _embed_body(x_ref, mf_ref, wfp_ref, u_ref, c0_ref, o_ref):
    x = x_ref[...]
    cnt = jnp.sum(mf_ref[...], axis=1, keepdims=True)
    o_ref[...] = (jnp.dot(x, wfp_ref[...], preferred_element_type=_f32)
                  + cnt * u_ref[...] + c0_ref[...])


def _embed(x, maskf, wfp, u, c0):
    bn = 2000
    return pl.pallas_call(
        _embed_body,
        grid=(N // bn,),
        in_specs=[
            pl.BlockSpec((bn, D), lambda i: (i, 0)),
            pl.BlockSpec((bn, D), lambda i: (i, 0)),
            pl.BlockSpec((D, HID), lambda i: (0, 0)),
            pl.BlockSpec((1, HID), lambda i: (0, 0)),
            pl.BlockSpec((1, HID), lambda i: (0, 0)),
        ],
        out_specs=pl.BlockSpec((bn, HID), lambda i: (i, 0)),
        out_shape=jax.ShapeDtypeStruct((N, HID), _f32),
    )(x, maskf, wfp, u, c0)


def _qkvs_body(h_ref, wq_ref, bq_ref, wk_ref, bk_ref, wv_ref, bv_ref,
               ws_ref, bs_ref, q_o, k_o, v_o, s_o):
    h = h_ref[...]
    q_o[...] = jnp.dot(h, wq_ref[...], preferred_element_type=_f32) + bq_ref[...]
    k_o[...] = jnp.dot(h, wk_ref[...], preferred_element_type=_f32) + bk_ref[...]
    v_o[...] = jnp.dot(h, wv_ref[...], preferred_element_type=_f32) + bv_ref[...]
    s_o[...] = jnp.dot(h, ws_ref[...], preferred_element_type=_f32) + bs_ref[...]


def _qkvs(h, wq, bq, wk, bk, wv, bv, ws, bs):
    bn = 2000
    wspec = pl.BlockSpec((HID, HID), lambda i: (0, 0))
    bspec = pl.BlockSpec((1, HID), lambda i: (0, 0))
    ospec = pl.BlockSpec((bn, HID), lambda i: (i, 0))
    oshape = jax.ShapeDtypeStruct((N, HID), _f32)
    return pl.pallas_call(
        _qkvs_body,
        grid=(N // bn,),
        in_specs=[pl.BlockSpec((bn, HID), lambda i: (i, 0)),
                  wspec, bspec, wspec, bspec, wspec, bspec, wspec, bspec],
        out_specs=[ospec, ospec, ospec, ospec],
        out_shape=[oshape, oshape, oshape, oshape],
    )(h, wq, bq, wk, bk, wv, bv, ws, bs)


NBKT = 640


_SC_MESH = dict(
    mesh=plsc.VectorSubcoreMesh(core_axis_name="c", subcore_axis_name="s",
                                num_cores=NSC),
    compiler_params=pltpu.CompilerParams(needs_layout_passes=False,
                                         use_tc_tiling_on_sc=False),
)


def _edge_a_body(q_hbm, k_hbm, src_hbm, dst_hbm, ea_hbm, outs_hbm,
                 srcv, dstv, bktv, qr, kr, ss, eab, zb, accs, sem0, sem1):
    c = lax.axis_index("c")
    s = lax.axis_index("s")
    wid = c * NSUB + s

    zero16 = jnp.zeros((16,), _f32)
    iota16 = lax.iota(jnp.int32, 16)

    def _zss(r, _):
        for j in range(HID // 16):
            ss[r, pl.ds(j * 16, 16)] = zero16
        return 0
    lax.fori_loop(0, CH, _zss, 0)

    def _zzb(r, _):
        for j in range(HID // 16):
            zb[r, pl.ds(j * 16, 16)] = zero16
        return 0
    lax.fori_loop(0, NBKT // NSUB, _zzb, 0)
    pltpu.sync_copy(zb.at[pl.ds(0, NBKT // NSUB)],
                    accs.at[pl.ds(s * (NBKT // NSUB), NBKT // NSUB)])
    plsc.subcore_barrier()

    def _chunk(i, _):
        ebase = wid * EPW + i * CH
        pltpu.sync_copy(src_hbm.at[pl.ds(ebase, CH)], srcv)
        pltpu.sync_copy(dst_hbm.at[pl.ds(ebase, CH)], dstv)
        cp_q = pltpu.async_copy(q_hbm.at[dstv], qr, sem0)
        cp_k = pltpu.async_copy(k_hbm.at[srcv], kr, sem1)

        def _bkt(g, _):
            dv = dstv[pl.ds(g * 16, 16)]
            bktv[pl.ds(g * 16, 16)] = lax.shift_right_logical(dv, 4)
            return 0
        lax.fori_loop(0, GRP, _bkt, 0)

        cp_q.wait()
        cp_k.wait()

        def _group(g, _):
            row_ids = g * 16 + iota16
            dv = dstv[pl.ds(g * 16, 16)]
            lane0 = (dv & 15) * 8
            for h in range(HEADS):
                acc_v = zero16
                for hc in range(HC):
                    col = jnp.full((16,), h * HC + hc, jnp.int32)
                    qv = plsc.load_gather(qr, [row_ids, col])
                    kv = plsc.load_gather(kr, [row_ids, col])
                    acc_v = acc_v + qv * kv
                ea = jnp.exp(acc_v * 0.25)
                plsc.store_scatter(eab, [row_ids, jnp.full((16,), h, jnp.int32)], ea)
                plsc.store_scatter(ss, [row_ids, lane0 + h], ea)
            return 0
        lax.fori_loop(0, GRP, _group, 0)

        pltpu.sync_copy(ss, accs.at[bktv], add=True)
        pltpu.sync_copy(eab, ea_hbm.at[pl.ds(ebase, CH)])

        def _zgroup(g, _):
            row_ids = g * 16 + iota16
            dv = dstv[pl.ds(g * 16, 16)]
            lane0 = (dv & 15) * 8
            for h in range(HEADS):
                plsc.store_scatter(ss, [row_ids, lane0 + h], zero16)
            return 0
        lax.fori_loop(0, GRP, _zgroup, 0)
        return 0

    lax.fori_loop(0, NCHUNK, _chunk, 0)

    plsc.subcore_barrier()
    pltpu.sync_copy(accs.at[pl.ds(s * (NBKT // NSUB), NBKT // NSUB)],
                    outs_hbm.at[c, pl.ds(s * (NBKT // NSUB), NBKT // NSUB)])


def _edge_b_body(v_hbm, ea_hbm, src_hbm, dst_hbm, outwv_hbm,
                 srcv, dstv, vr, swv, eab, zb, accwv, sem0):
    c = lax.axis_index("c")
    s = lax.axis_index("s")
    wid = c * NSUB + s

    zero16 = jnp.zeros((16,), _f32)

    def _zzb(r, _):
        for j in range(HID // 16):
            zb[r, pl.ds(j * 16, 16)] = zero16
        return 0
    lax.fori_loop(0, 128, _zzb, 0)
    for t in range(5):
        pltpu.sync_copy(zb.at[pl.ds(0, 128)],
                        accwv.at[pl.ds(s * 624 + t * 128, 128)])
    plsc.subcore_barrier()

    def _chunk(i, _):
        ebase = wid * EPW + i * CH
        pltpu.sync_copy(src_hbm.at[pl.ds(ebase, CH)], srcv)
        pltpu.sync_copy(dst_hbm.at[pl.ds(ebase, CH)], dstv)
        cp_v = pltpu.async_copy(v_hbm.at[srcv], vr, sem0)
        pltpu.sync_copy(ea_hbm.at[pl.ds(ebase, CH)], eab)
        cp_v.wait()

        def _group(g, _):
            for e in range(16):
                r = g * 16 + e
                for h in range(HEADS):
                    b = plsc.load_gather(
                        eab, [jnp.full((16,), r, jnp.int32),
                              jnp.full((16,), h, jnp.int32)])
                    vv = vr[r, pl.ds(h * HC, 16)]
                    swv[r, pl.ds(h * HC, 16)] = b * vv
            return 0
        lax.fori_loop(0, GRP, _group, 0)

        pltpu.sync_copy(swv, accwv.at[dstv], add=True)
        return 0

    lax.fori_loop(0, NCHUNK, _chunk, 0)

    plsc.subcore_barrier()

    @pl.when(s < NSUB - 1)
    def _copy_main():
        pltpu.sync_copy(accwv.at[pl.ds(s * 624, 624)],
                        outwv_hbm.at[c, pl.ds(s * 624, 624)])

    @pl.when(s == NSUB - 1)
    def _copy_tail():
        pltpu.sync_copy(accwv.at[pl.ds((NSUB - 1) * 624, N - (NSUB - 1) * 624)],
                        outwv_hbm.at[c, pl.ds((NSUB - 1) * 624, N - (NSUB - 1) * 624)])


def _edge_phase(q, k, v, src, dst):
    ea, outs = pl.kernel(
        _edge_a_body,
        out_type=[jax.ShapeDtypeStruct((E, HEADS), _f32),
                  jax.ShapeDtypeStruct((NSC, NBKT, HID), _f32)],
        scratch_types=[
            pltpu.VMEM((CH,), jnp.int32),
            pltpu.VMEM((CH,), jnp.int32),
            pltpu.VMEM((CH,), jnp.int32),
            pltpu.VMEM((CH, HID), _f32),
            pltpu.VMEM((CH, HID), _f32),
            pltpu.VMEM((CH, HID), _f32),
            pltpu.VMEM((CH, HEADS), _f32),
            pltpu.VMEM((NBKT // NSUB, HID), _f32),
            pltpu.VMEM_SHARED((NBKT, HID), _f32),
            pltpu.SemaphoreType.DMA,
            pltpu.SemaphoreType.DMA,
        ],
        **_SC_MESH,
    )(q, k, src, dst)

    outwv = pl.kernel(
        _edge_b_body,
        out_type=jax.ShapeDtypeStruct((NSC, N, HID), _f32),
        scratch_types=[
            pltpu.VMEM((CH,), jnp.int32),
            pltpu.VMEM((CH,), jnp.int32),
            pltpu.VMEM((CH, HID), _f32),
            pltpu.VMEM((CH, HID), _f32),
            pltpu.VMEM((CH, HEADS), _f32),
            pltpu.VMEM((128, HID), _f32),
            pltpu.VMEM_SHARED((N, HID), _f32),
            pltpu.SemaphoreType.DMA,
        ],
        **_SC_MESH,
    )(v, ea, src, dst)
    return outwv, outs


def _combine_body(s2_ref, wv2_ref, sh_ref, hr_ref, g_ref, b_ref, rep_ref,
                  o_ref, *, relu):
    svec = jnp.sum(s2_ref[...], axis=0)
    wv = jnp.sum(wv2_ref[...], axis=0)
    srep = jnp.dot(1.0 / (svec + 1e-16), rep_ref[...],
                   preferred_element_type=_f32)
    t = wv * srep + sh_ref[...] + hr_ref[...]
    mu = jnp.mean(t, axis=-1, keepdims=True)
    var = jnp.mean((t - mu) ** 2, axis=-1, keepdims=True)
    y = (t - mu) * lax.rsqrt(var + 1e-5) * g_ref[...] + b_ref[...]
    if relu:
        y = jnp.maximum(y, 0.0)
    o_ref[...] = y


def _combine(s2, wv2, sh, hres, g, b, rep, relu):
    bn = 2000
    body = functools.partial(_combine_body, relu=relu)
    return pl.pallas_call(
        body,
        grid=(N // bn,),
        in_specs=[
            pl.BlockSpec((NSC, bn, HEADS), lambda i: (0, i, 0)),
            pl.BlockSpec((NSC, bn, HID), lambda i: (0, i, 0)),
            pl.BlockSpec((bn, HID), lambda i: (i, 0)),
            pl.BlockSpec((bn, HID), lambda i: (i, 0)),
            pl.BlockSpec((1, HID), lambda i: (0, 0)),
            pl.BlockSpec((1, HID), lambda i: (0, 0)),
            pl.BlockSpec((HEADS, HID), lambda i: (0, 0)),
        ],
        out_specs=pl.BlockSpec((bn, HID), lambda i: (i, 0)),
        out_shape=jax.ShapeDtypeStruct((N, HID), _f32),
    )(s2, wv2, sh, hres, g, b, rep)


def _heads_body(h_ref, p1w_ref, p1b_ref, p2w_ref, p2b_ref, p3w_ref, p3b_ref,
                u1w_ref, u1b_ref, u2w_ref, u2b_ref, g1w_ref, g1b_ref,
                g2w_ref, g2b_ref, preds_o, unc_o, gc_o):
    h = h_ref[...]
    preds_cols = []
    unc_cols = []
    for d in range(D):
        a1 = jnp.maximum(jnp.dot(h, p1w_ref[d], preferred_element_type=_f32)
                         + p1b_ref[d][None, :], 0.0)
        a2 = jnp.maximum(jnp.dot(a1, p2w_ref[d], preferred_element_type=_f32)
                         + p2b_ref[d][None, :], 0.0)
        preds_cols.append(jnp.dot(a2, p3w_ref[d], preferred_element_type=_f32)
                          + p3b_ref[d][None, :])
        u1 = jnp.maximum(jnp.dot(h, u1w_ref[d], preferred_element_type=_f32)
                         + u1b_ref[d][None, :], 0.0)
        u2 = (jnp.dot(u1, u2w_ref[d], preferred_element_type=_f32)
              + u2b_ref[d][None, :])
        um = jnp.minimum(u2, 20.0)
        unc_cols.append(jnp.where(u2 > 20.0, u2,
                                  jnp.log(1.0 + jnp.exp(um))))
    preds_o[...] = jnp.concatenate(preds_cols, axis=1)
    unc_o[...] = jnp.concatenate(unc_cols, axis=1)
    gl = jnp.maximum(jnp.dot(h, g1w_ref[...], preferred_element_type=_f32)
                     + g1b_ref[...], 0.0)
    gl = jnp.dot(gl, g2w_ref[...], preferred_element_type=_f32) + g2b_ref[...]
    gc_o[...] = 1.0 / (1.0 + jnp.exp(-gl))


def _heads(h, P1W, P1b, P2W, P2b, P3W, P3b, U1W, U1b, U2W, U2b,
           G1W, G1b, G2W, G2b):
    bn = 1000
    full = lambda shape: pl.BlockSpec(shape, lambda i: tuple(0 for _ in shape))
    return pl.pallas_call(
        _heads_body,
        grid=(N // bn,),
        in_specs=[
            pl.BlockSpec((bn, HID), lambda i: (i, 0)),
            full((D, HID, HID // 2)), full((D, HID // 2)),
            full((D, HID // 2, HID // 4)), full((D, HID // 4)),
            full((D, HID // 4, 1)), full((D, 1)),
            full((D, HID, HID // 4)), full((D, HID // 4)),
            full((D, HID // 4, 1)), full((D, 1)),
            full((HID, HID // 2)), full((1, HID // 2)),
            full((HID // 2, 1)), full((1, 1)),
        ],
        out_specs=[pl.BlockSpec((bn, D), lambda i: (i, 0)),
                   pl.BlockSpec((bn, D), lambda i: (i, 0)),
                   pl.BlockSpec((bn, 1), lambda i: (i, 0))],
        out_shape=[jax.ShapeDtypeStruct((N, D), _f32),
                   jax.ShapeDtypeStruct((N, D), _f32),
                   jax.ShapeDtypeStruct((N, 1), _f32)],
    )(h, P1W, P1b, P2W, P2b, P3W, P3b, U1W, U1b, U2W, U2b,
      G1W, G1b[None, :], G2W, G2b[None, :])


def kernel(x, edge_index, missing_mask, Wf, bf, Emiss, Etype, Wp, bp,
           Wq, bq, Wk, bk, Wv, bv, Ws, bs, ln_g, ln_b,
           P1W, P1b, P2W, P2b, P3W, P3b, U1W, U1b, U2W, U2b,
           G1W, G1b, G2W, G2b):
    Wp0, Wp1, Wp2 = Wp[:Q], Wp[Q:2 * Q], Wp[2 * Q:]
    wfp = (Wf @ Wp0) / D
    u = (((Emiss[1] - Emiss[0]) @ Wp1) / D)[None, :]
    c0 = (bf.mean(0) @ Wp0 + Emiss[0] @ Wp1 + Etype[0] @ Wp2 + bp)[None, :]
    maskf = missing_mask.astype(_f32)

    h = _embed(x, maskf, wfp, u, c0)

    src = edge_index[0]
    dst = edge_index[1]
    rep = jnp.repeat(jnp.eye(HEADS, dtype=_f32), HC, axis=1)

    for l in range(NL):
        hres = h
        q, k, v, sh = _qkvs(h, Wq[l], bq[l][None, :], Wk[l], bk[l][None, :],
                            Wv[l], bv[l][None, :], Ws[l], bs[l][None, :])
        wv2, aggs = _edge_phase(q, k, v, src, dst)
        s2 = aggs.reshape(NSC, NBKT * HC, HEADS)[:, :N]
        h = _combine(s2, wv2, sh, hres, ln_g[l][None, :], ln_b[l][None, :],
                     rep, relu=(l < NL - 1))

    preds, unc, gc = _heads(h, P1W, P1b, P2W, P2b, P3W, P3b,
                            U1W, U1b, U2W, U2b, G1W, G1b, G2W, G2b)
    return (preds, unc, gc)


# speedup vs baseline: 27.3400x; 1.1423x over previous
import functools

import jax
import jax.numpy as jnp
from jax import lax
from jax.experimental import pallas as pl
from jax.experimental.pallas import tpu as pltpu
from jax.experimental.pallas import tpu_sc as plsc

N = 10000
D = 32
HID = 128
HEADS = 8
HC = 16
E = 640000
NL = 4
Q = HID // 4

NSC = 2
NSUB = 16
NW = NSC * NSUB
EPW = E // NW
CH = 80
NCHUNK = EPW // CH
GRP = CH // 16

_f32 = jnp.float32


def _embed_body(x_ref, mf_ref, wfp_ref, u_ref, c0_ref, o_ref):
    x = x_ref[...]
    cnt = jnp.sum(mf_ref[...], axis=1, keepdims=True)
    o_ref[...] = (jnp.dot(x, wfp_ref[...], preferred_element_type=_f32)
                  + cnt * u_ref[...] + c0_ref[...])


def _embed(x, maskf, wfp, u, c0):
    bn = 2000
    return pl.pallas_call(
        _embed_body,
        grid=(N // bn,),
        in_specs=[
            pl.BlockSpec((bn, D), lambda i: (i, 0)),
            pl.BlockSpec((bn, D), lambda i: (i, 0)),
            pl.BlockSpec((D, HID), lambda i: (0, 0)),
            pl.BlockSpec((1, HID), lambda i: (0, 0)),
            pl.BlockSpec((1, HID), lambda i: (0, 0)),
        ],
        out_specs=pl.BlockSpec((bn, HID), lambda i: (i, 0)),
        out_shape=jax.ShapeDtypeStruct((N, HID), _f32),
    )(x, maskf, wfp, u, c0)


def _qkvs_body(h_ref, wq_ref, bq_ref, wk_ref, bk_ref, wv_ref, bv_ref,
               ws_ref, bs_ref, q_o, k_o, v_o, s_o):
    h = h_ref[...]
    q_o[...] = jnp.dot(h, wq_ref[...], preferred_element_type=_f32) + bq_ref[...]
    k_o[...] = jnp.dot(h, wk_ref[...], preferred_element_type=_f32) + bk_ref[...]
    v_o[...] = jnp.dot(h, wv_ref[...], preferred_element_type=_f32) + bv_ref[...]
    s_o[...] = jnp.dot(h, ws_ref[...], preferred_element_type=_f32) + bs_ref[...]


def _qkvs(h, wq, bq, wk, bk, wv, bv, ws, bs):
    bn = 2000
    wspec = pl.BlockSpec((HID, HID), lambda i: (0, 0))
    bspec = pl.BlockSpec((1, HID), lambda i: (0, 0))
    ospec = pl.BlockSpec((bn, HID), lambda i: (i, 0))
    oshape = jax.ShapeDtypeStruct((N, HID), _f32)
    return pl.pallas_call(
        _qkvs_body,
        grid=(N // bn,),
        in_specs=[pl.BlockSpec((bn, HID), lambda i: (i, 0)),
                  wspec, bspec, wspec, bspec, wspec, bspec, wspec, bspec],
        out_specs=[ospec, ospec, ospec, ospec],
        out_shape=[oshape, oshape, oshape, oshape],
    )(h, wq, bq, wk, bk, wv, bv, ws, bs)


NBKT = 640


_SC_MESH = dict(
    mesh=plsc.VectorSubcoreMesh(core_axis_name="c", subcore_axis_name="s",
                                num_cores=NSC),
    compiler_params=pltpu.CompilerParams(needs_layout_passes=False,
                                         use_tc_tiling_on_sc=False),
)


def _edge_a_body(q_hbm, k_hbm, src_hbm, dst_hbm, ea_hbm, outs_hbm,
                 srcv0, srcv1, dstv0, dstv1, bktv, qr0, qr1, kr0, kr1,
                 ss, eab, zb, accs, semq0, semq1, semk0, semk1):
    c = lax.axis_index("c")
    s = lax.axis_index("s")
    wid = c * NSUB + s
    srcv = (srcv0, srcv1)
    dstv = (dstv0, dstv1)
    qr = (qr0, qr1)
    kr = (kr0, kr1)
    semq = (semq0, semq1)
    semk = (semk0, semk1)

    zero16 = jnp.zeros((16,), _f32)
    iota16 = lax.iota(jnp.int32, 16)

    def _zss(r, _):
        for j in range(HID // 16):
            ss[r, pl.ds(j * 16, 16)] = zero16
        return 0
    lax.fori_loop(0, CH, _zss, 0)

    def _zzb(r, _):
        for j in range(HID // 16):
            zb[r, pl.ds(j * 16, 16)] = zero16
        return 0
    lax.fori_loop(0, NBKT // NSUB, _zzb, 0)
    pltpu.sync_copy(zb.at[pl.ds(0, NBKT // NSUB)],
                    accs.at[pl.ds(s * (NBKT // NSUB), NBKT // NSUB)])
    plsc.subcore_barrier()

    def _issue(ci, b):
        ebase = wid * EPW + ci * CH
        pltpu.sync_copy(src_hbm.at[pl.ds(ebase, CH)], srcv[b])
        pltpu.sync_copy(dst_hbm.at[pl.ds(ebase, CH)], dstv[b])
        pltpu.async_copy(q_hbm.at[dstv[b]], qr[b], semq[b])
        pltpu.async_copy(k_hbm.at[srcv[b]], kr[b], semk[b])

    _issue(0, 0)

    def _outer(i, _):
        for b in range(2):
            ci = 2 * i + b

            @pl.when(ci + 1 < NCHUNK)
            def _():
                _issue(ci + 1, 1 - b)

            pltpu.make_async_copy(q_hbm.at[dstv[b]], qr[b], semq[b]).wait()
            pltpu.make_async_copy(k_hbm.at[srcv[b]], kr[b], semk[b]).wait()

            def _group(g, _, b=b):
                row_ids = g * 16 + iota16
                dv = dstv[b][pl.ds(g * 16, 16)]
                lane0 = (dv & 15) * 8
                bktv[pl.ds(g * 16, 16)] = lax.shift_right_logical(dv, 4)
                for h in range(HEADS):
                    acc_v = zero16
                    for hc in range(HC):
                        col = jnp.full((16,), h * HC + hc, jnp.int32)
                        qv = plsc.load_gather(qr[b], [row_ids, col])
                        kv = plsc.load_gather(kr[b], [row_ids, col])
                        acc_v = acc_v + qv * kv
                    ea = jnp.exp(acc_v * 0.25)
                    plsc.store_scatter(
                        eab, [row_ids, jnp.full((16,), h, jnp.int32)], ea)
                    plsc.store_scatter(ss, [row_ids, lane0 + h], ea)
                return 0
            lax.fori_loop(0, GRP, _group, 0)

            ebase = wid * EPW + ci * CH
            pltpu.sync_copy(ss, accs.at[bktv], add=True)
            pltpu.sync_copy(eab, ea_hbm.at[pl.ds(ebase, CH)])

            def _zgroup(g, _, b=b):
                row_ids = g * 16 + iota16
                dv = dstv[b][pl.ds(g * 16, 16)]
                lane0 = (dv & 15) * 8
                for h in range(HEADS):
                    plsc.store_scatter(ss, [row_ids, lane0 + h], zero16)
                return 0
            lax.fori_loop(0, GRP, _zgroup, 0)
        return 0

    lax.fori_loop(0, NCHUNK // 2, _outer, 0)

    plsc.subcore_barrier()
    pltpu.sync_copy(accs.at[pl.ds(s * (NBKT // NSUB), NBKT // NSUB)],
                    outs_hbm.at[c, pl.ds(s * (NBKT // NSUB), NBKT // NSUB)])


def _edge_b_body(v_hbm, ea_hbm, src_hbm, dst_hbm, outwv_hbm,
                 srcv0, srcv1, dstv0, dstv1, vr0, vr1, eab0, eab1,
                 swv, zb, accwv, semv0, semv1, seme0, seme1):
    c = lax.axis_index("c")
    s = lax.axis_index("s")
    wid = c * NSUB + s
    srcv = (srcv0, srcv1)
    dstv = (dstv0, dstv1)
    vr = (vr0, vr1)
    eab = (eab0, eab1)
    semv = (semv0, semv1)
    seme = (seme0, seme1)

    zero16 = jnp.zeros((16,), _f32)

    def _zzb(r, _):
        for j in range(HID // 16):
            zb[r, pl.ds(j * 16, 16)] = zero16
        return 0
    lax.fori_loop(0, 128, _zzb, 0)
    for t in range(5):
        pltpu.sync_copy(zb.at[pl.ds(0, 128)],
                        accwv.at[pl.ds(s * 624 + t * 128, 128)])
    plsc.subcore_barrier()

    def _issue(ci, b):
        ebase = wid * EPW + ci * CH
        pltpu.sync_copy(src_hbm.at[pl.ds(ebase, CH)], srcv[b])
        pltpu.sync_copy(dst_hbm.at[pl.ds(ebase, CH)], dstv[b])
        pltpu.async_copy(v_hbm.at[srcv[b]], vr[b], semv[b])
        pltpu.async_copy(ea_hbm.at[pl.ds(ebase, CH)], eab[b], seme[b])

    _issue(0, 0)

    def _outer(i, _):
        for b in range(2):
            ci = 2 * i + b

            @pl.when(ci + 1 < NCHUNK)
            def _():
                _issue(ci + 1, 1 - b)

            ebase = wid * EPW + ci * CH
            pltpu.make_async_copy(v_hbm.at[srcv[b]], vr[b], semv[b]).wait()
            pltpu.make_async_copy(ea_hbm.at[pl.ds(ebase, CH)], eab[b],
                                  seme[b]).wait()

            def _group(g, _, b=b):
                for e in range(16):
                    r = g * 16 + e
                    for h in range(HEADS):
                        bv = plsc.load_gather(
                            eab[b], [jnp.full((16,), r, jnp.int32),
                                     jnp.full((16,), h, jnp.int32)])
                        vv = vr[b][r, pl.ds(h * HC, 16)]
                        swv[r, pl.ds(h * HC, 16)] = bv * vv
                return 0
            lax.fori_loop(0, GRP, _group, 0)

            pltpu.sync_copy(swv, accwv.at[dstv[b]], add=True)
        return 0

    lax.fori_loop(0, NCHUNK // 2, _outer, 0)

    plsc.subcore_barrier()

    @pl.when(s < NSUB - 1)
    def _copy_main():
        pltpu.sync_copy(accwv.at[pl.ds(s * 624, 624)],
                        outwv_hbm.at[c, pl.ds(s * 624, 624)])

    @pl.when(s == NSUB - 1)
    def _copy_tail():
        pltpu.sync_copy(accwv.at[pl.ds((NSUB - 1) * 624, N - (NSUB - 1) * 624)],
                        outwv_hbm.at[c, pl.ds((NSUB - 1) * 624, N - (NSUB - 1) * 624)])


def _edge_phase(q, k, v, src, dst):
    ea, outs = pl.kernel(
        _edge_a_body,
        out_type=[jax.ShapeDtypeStruct((E, HEADS), _f32),
                  jax.ShapeDtypeStruct((NSC, NBKT, HID), _f32)],
        scratch_types=[
            pltpu.VMEM((CH,), jnp.int32),
            pltpu.VMEM((CH,), jnp.int32),
            pltpu.VMEM((CH,), jnp.int32),
            pltpu.VMEM((CH,), jnp.int32),
            pltpu.VMEM((CH,), jnp.int32),
            pltpu.VMEM((CH, HID), _f32),
            pltpu.VMEM((CH, HID), _f32),
            pltpu.VMEM((CH, HID), _f32),
            pltpu.VMEM((CH, HID), _f32),
            pltpu.VMEM((CH, HID), _f32),
            pltpu.VMEM((CH, HEADS), _f32),
            pltpu.VMEM((NBKT // NSUB, HID), _f32),
            pltpu.VMEM_SHARED((NBKT, HID), _f32),
            pltpu.SemaphoreType.DMA,
            pltpu.SemaphoreType.DMA,
            pltpu.SemaphoreType.DMA,
            pltpu.SemaphoreType.DMA,
        ],
        **_SC_MESH,
    )(q, k, src, dst)

    outwv = pl.kernel(
        _edge_b_body,
        out_type=jax.ShapeDtypeStruct((NSC, N, HID), _f32),
        scratch_types=[
            pltpu.VMEM((CH,), jnp.int32),
            pltpu.VMEM((CH,), jnp.int32),
            pltpu.VMEM((CH,), jnp.int32),
            pltpu.VMEM((CH,), jnp.int32),
            pltpu.VMEM((CH, HID), _f32),
            pltpu.VMEM((CH, HID), _f32),
            pltpu.VMEM((CH, HEADS), _f32),
            pltpu.VMEM((CH, HEADS), _f32),
            pltpu.VMEM((CH, HID), _f32),
            pltpu.VMEM((128, HID), _f32),
            pltpu.VMEM_SHARED((N, HID), _f32),
            pltpu.SemaphoreType.DMA,
            pltpu.SemaphoreType.DMA,
            pltpu.SemaphoreType.DMA,
            pltpu.SemaphoreType.DMA,
        ],
        **_SC_MESH,
    )(v, ea, src, dst)
    return outwv, outs


def _combine_body(s2_ref, wv2_ref, sh_ref, hr_ref, g_ref, b_ref, rep_ref,
                  o_ref, *, relu):
    svec = jnp.sum(s2_ref[...], axis=0)
    wv = jnp.sum(wv2_ref[...], axis=0)
    srep = jnp.dot(1.0 / (svec + 1e-16), rep_ref[...],
                   preferred_element_type=_f32)
    t = wv * srep + sh_ref[...] + hr_ref[...]
    mu = jnp.mean(t, axis=-1, keepdims=True)
    var = jnp.mean((t - mu) ** 2, axis=-1, keepdims=True)
    y = (t - mu) * lax.rsqrt(var + 1e-5) * g_ref[...] + b_ref[...]
    if relu:
        y = jnp.maximum(y, 0.0)
    o_ref[...] = y


def _combine(s2, wv2, sh, hres, g, b, rep, relu):
    bn = 2000
    body = functools.partial(_combine_body, relu=relu)
    return pl.pallas_call(
        body,
        grid=(N // bn,),
        in_specs=[
            pl.BlockSpec((NSC, bn, HEADS), lambda i: (0, i, 0)),
            pl.BlockSpec((NSC, bn, HID), lambda i: (0, i, 0)),
            pl.BlockSpec((bn, HID), lambda i: (i, 0)),
            pl.BlockSpec((bn, HID), lambda i: (i, 0)),
            pl.BlockSpec((1, HID), lambda i: (0, 0)),
            pl.BlockSpec((1, HID), lambda i: (0, 0)),
            pl.BlockSpec((HEADS, HID), lambda i: (0, 0)),
        ],
        out_specs=pl.BlockSpec((bn, HID), lambda i: (i, 0)),
        out_shape=jax.ShapeDtypeStruct((N, HID), _f32),
    )(s2, wv2, sh, hres, g, b, rep)


def _heads_body(h_ref, p1w_ref, p1b_ref, p2w_ref, p2b_ref, p3w_ref, p3b_ref,
                u1w_ref, u1b_ref, u2w_ref, u2b_ref, g1w_ref, g1b_ref,
                g2w_ref, g2b_ref, preds_o, unc_o, gc_o):
    h = h_ref[...]
    preds_cols = []
    unc_cols = []
    for d in range(D):
        a1 = jnp.maximum(jnp.dot(h, p1w_ref[d], preferred_element_type=_f32)
                         + p1b_ref[d][None, :], 0.0)
        a2 = jnp.maximum(jnp.dot(a1, p2w_ref[d], preferred_element_type=_f32)
                         + p2b_ref[d][None, :], 0.0)
        preds_cols.append(jnp.dot(a2, p3w_ref[d], preferred_element_type=_f32)
                          + p3b_ref[d][None, :])
        u1 = jnp.maximum(jnp.dot(h, u1w_ref[d], preferred_element_type=_f32)
                         + u1b_ref[d][None, :], 0.0)
        u2 = (jnp.dot(u1, u2w_ref[d], preferred_element_type=_f32)
              + u2b_ref[d][None, :])
        um = jnp.minimum(u2, 20.0)
        unc_cols.append(jnp.where(u2 > 20.0, u2,
                                  jnp.log(1.0 + jnp.exp(um))))
    preds_o[...] = jnp.concatenate(preds_cols, axis=1)
    unc_o[...] = jnp.concatenate(unc_cols, axis=1)
    gl = jnp.maximum(jnp.dot(h, g1w_ref[...], preferred_element_type=_f32)
                     + g1b_ref[...], 0.0)
    gl = jnp.dot(gl, g2w_ref[...], preferred_element_type=_f32) + g2b_ref[...]
    gc_o[...] = 1.0 / (1.0 + jnp.exp(-gl))


def _heads(h, P1W, P1b, P2W, P2b, P3W, P3b, U1W, U1b, U2W, U2b,
           G1W, G1b, G2W, G2b):
    bn = 1000
    full = lambda shape: pl.BlockSpec(shape, lambda i: tuple(0 for _ in shape))
    return pl.pallas_call(
        _heads_body,
        grid=(N // bn,),
        in_specs=[
            pl.BlockSpec((bn, HID), lambda i: (i, 0)),
            full((D, HID, HID // 2)), full((D, HID // 2)),
            full((D, HID // 2, HID // 4)), full((D, HID // 4)),
            full((D, HID // 4, 1)), full((D, 1)),
            full((D, HID, HID // 4)), full((D, HID // 4)),
            full((D, HID // 4, 1)), full((D, 1)),
            full((HID, HID // 2)), full((1, HID // 2)),
            full((HID // 2, 1)), full((1, 1)),
        ],
        out_specs=[pl.BlockSpec((bn, D), lambda i: (i, 0)),
                   pl.BlockSpec((bn, D), lambda i: (i, 0)),
                   pl.BlockSpec((bn, 1), lambda i: (i, 0))],
        out_shape=[jax.ShapeDtypeStruct((N, D), _f32),
                   jax.ShapeDtypeStruct((N, D), _f32),
                   jax.ShapeDtypeStruct((N, 1), _f32)],
    )(h, P1W, P1b, P2W, P2b, P3W, P3b, U1W, U1b, U2W, U2b,
      G1W, G1b[None, :], G2W, G2b[None, :])


def kernel(x, edge_index, missing_mask, Wf, bf, Emiss, Etype, Wp, bp,
           Wq, bq, Wk, bk, Wv, bv, Ws, bs, ln_g, ln_b,
           P1W, P1b, P2W, P2b, P3W, P3b, U1W, U1b, U2W, U2b,
           G1W, G1b, G2W, G2b):
    Wp0, Wp1, Wp2 = Wp[:Q], Wp[Q:2 * Q], Wp[2 * Q:]
    wfp = (Wf @ Wp0) / D
    u = (((Emiss[1] - Emiss[0]) @ Wp1) / D)[None, :]
    c0 = (bf.mean(0) @ Wp0 + Emiss[0] @ Wp1 + Etype[0] @ Wp2 + bp)[None, :]
    maskf = missing_mask.astype(_f32)

    h = _embed(x, maskf, wfp, u, c0)

    src = edge_index[0]
    dst = edge_index[1]
    rep = jnp.repeat(jnp.eye(HEADS, dtype=_f32), HC, axis=1)

    for l in range(NL):
        hres = h
        q, k, v, sh = _qkvs(h, Wq[l], bq[l][None, :], Wk[l], bk[l][None, :],
                            Wv[l], bv[l][None, :], Ws[l], bs[l][None, :])
        wv2, aggs = _edge_phase(q, k, v, src, dst)
        s2 = aggs.reshape(NSC, NBKT * HC, HEADS)[:, :N]
        h = _combine(s2, wv2, sh, hres, ln_g[l][None, :], ln_b[l][None, :],
                     rep, relu=(l < NL - 1))

    preds, unc, gc = _heads(h, P1W, P1b, P2W, P2b, P3W, P3b,
                            U1W, U1b, U2W, U2b, G1W, G1b, G2W, G2b)
    return (preds, unc, gc)


# speedup vs baseline: 60.0532x; 2.1965x over previous
import functools

import jax
import jax.numpy as jnp
from jax import lax
from jax.experimental import pallas as pl
from jax.experimental.pallas import tpu as pltpu
from jax.experimental.pallas import tpu_sc as plsc

N = 10000
D = 32
HID = 128
HEADS = 8
HC = 16
E = 640000
NL = 4
Q = HID // 4

NSC = 2
NSUB = 16
NW = NSC * NSUB
EPW = E // NW
CH = 80
NCHUNK = EPW // CH
GRP = CH // 16

_f32 = jnp.float32


def _embed_body(x_ref, mf_ref, wfp_ref, u_ref, c0_ref, o_ref):
    x = x_ref[...]
    cnt = jnp.sum(mf_ref[...], axis=1, keepdims=True)
    o_ref[...] = (jnp.dot(x, wfp_ref[...], preferred_element_type=_f32)
                  + cnt * u_ref[...] + c0_ref[...])


def _embed(x, maskf, wfp, u, c0):
    bn = 2000
    return pl.pallas_call(
        _embed_body,
        grid=(N // bn,),
        in_specs=[
            pl.BlockSpec((bn, D), lambda i: (i, 0)),
            pl.BlockSpec((bn, D), lambda i: (i, 0)),
            pl.BlockSpec((D, HID), lambda i: (0, 0)),
            pl.BlockSpec((1, HID), lambda i: (0, 0)),
            pl.BlockSpec((1, HID), lambda i: (0, 0)),
        ],
        out_specs=pl.BlockSpec((bn, HID), lambda i: (i, 0)),
        out_shape=jax.ShapeDtypeStruct((N, HID), _f32),
    )(x, maskf, wfp, u, c0)


def _qkvs_body(h_ref, wq_ref, bq_ref, wk_ref, bk_ref, wv_ref, bv_ref,
               ws_ref, bs_ref, q_o, k_o, v_o, s_o):
    h = h_ref[...]
    q_o[...] = jnp.dot(h, wq_ref[...], preferred_element_type=_f32) + bq_ref[...]
    k_o[...] = jnp.dot(h, wk_ref[...], preferred_element_type=_f32) + bk_ref[...]
    v_o[...] = jnp.dot(h, wv_ref[...], preferred_element_type=_f32) + bv_ref[...]
    s_o[...] = jnp.dot(h, ws_ref[...], preferred_element_type=_f32) + bs_ref[...]


def _qkvs(h, wq, bq, wk, bk, wv, bv, ws, bs):
    bn = 2000
    wspec = pl.BlockSpec((HID, HID), lambda i: (0, 0))
    bspec = pl.BlockSpec((1, HID), lambda i: (0, 0))
    ospec = pl.BlockSpec((bn, HID), lambda i: (i, 0))
    oshape = jax.ShapeDtypeStruct((N, HID), _f32)
    return pl.pallas_call(
        _qkvs_body,
        grid=(N // bn,),
        in_specs=[pl.BlockSpec((bn, HID), lambda i: (i, 0)),
                  wspec, bspec, wspec, bspec, wspec, bspec, wspec, bspec],
        out_specs=[ospec, ospec, ospec, ospec],
        out_shape=[oshape, oshape, oshape, oshape],
    )(h, wq, bq, wk, bk, wv, bv, ws, bs)


NBKT = 640


_SC_MESH = dict(
    mesh=plsc.VectorSubcoreMesh(core_axis_name="c", subcore_axis_name="s",
                                num_cores=NSC),
    compiler_params=pltpu.CompilerParams(needs_layout_passes=False,
                                         use_tc_tiling_on_sc=False),
)


def _edge_a_body(q_hbm, k_hbm, src_hbm, dst_hbm, ea_hbm, outs_hbm,
                 srcv0, srcv1, dstv0, dstv1, bktv, qr0, qr1, kr0, kr1,
                 ss, eab, zb, accs, semq0, semq1, semk0, semk1):
    c = lax.axis_index("c")
    s = lax.axis_index("s")
    wid = c * NSUB + s
    srcv = (srcv0, srcv1)
    dstv = (dstv0, dstv1)
    qr = (qr0, qr1)
    kr = (kr0, kr1)
    semq = (semq0, semq1)
    semk = (semk0, semk1)

    zero16 = jnp.zeros((16,), _f32)
    iota16 = lax.iota(jnp.int32, 16)

    def _zss(r, _):
        for j in range(HID // 16):
            ss[r, pl.ds(j * 16, 16)] = zero16
        return 0
    lax.fori_loop(0, CH, _zss, 0)

    def _zzb(r, _):
        for j in range(HID // 16):
            zb[r, pl.ds(j * 16, 16)] = zero16
        return 0
    lax.fori_loop(0, NBKT // NSUB, _zzb, 0)
    pltpu.sync_copy(zb.at[pl.ds(0, NBKT // NSUB)],
                    accs.at[pl.ds(s * (NBKT // NSUB), NBKT // NSUB)])
    plsc.subcore_barrier()

    def _issue(ci, b):
        ebase = wid * EPW + ci * CH
        pltpu.sync_copy(src_hbm.at[pl.ds(ebase, CH)], srcv[b])
        pltpu.sync_copy(dst_hbm.at[pl.ds(ebase, CH)], dstv[b])
        pltpu.async_copy(q_hbm.at[dstv[b]], qr[b], semq[b])
        pltpu.async_copy(k_hbm.at[srcv[b]], kr[b], semk[b])

    _issue(0, 0)

    def _outer(i, _):
        for b in range(2):
            ci = 2 * i + b

            @pl.when(ci + 1 < NCHUNK)
            def _():
                _issue(ci + 1, 1 - b)

            pltpu.make_async_copy(q_hbm.at[dstv[b]], qr[b], semq[b]).wait()
            pltpu.make_async_copy(k_hbm.at[srcv[b]], kr[b], semk[b]).wait()

            def _group(g, _, b=b):
                row_ids = g * 16 + iota16
                dv = dstv[b][pl.ds(g * 16, 16)]
                lane0 = (dv & 15) * 8
                bktv[pl.ds(g * 16, 16)] = lax.shift_right_logical(dv, 4)
                for h in range(HEADS):
                    acc_v = zero16
                    for j in range(HC):
                        col = h * HC + ((iota16 + (j + g)) & 15)
                        qv = plsc.load_gather(qr[b], [row_ids, col])
                        kv = plsc.load_gather(kr[b], [row_ids, col])
                        acc_v = acc_v + qv * kv
                    ea = jnp.exp(acc_v * 0.25)
                    eab[h, pl.ds(g * 16, 16)] = ea
                    plsc.store_scatter(ss, [row_ids, lane0 + h], ea)
                return 0
            lax.fori_loop(0, GRP, _group, 0)

            ebase = wid * EPW + ci * CH
            pltpu.sync_copy(ss, accs.at[bktv], add=True)
            pltpu.sync_copy(eab, ea_hbm.at[pl.ds(0, HEADS), pl.ds(ebase, CH)])

            def _zgroup(g, _, b=b):
                row_ids = g * 16 + iota16
                dv = dstv[b][pl.ds(g * 16, 16)]
                lane0 = (dv & 15) * 8
                for h in range(HEADS):
                    plsc.store_scatter(ss, [row_ids, lane0 + h], zero16)
                return 0
            lax.fori_loop(0, GRP, _zgroup, 0)
        return 0

    lax.fori_loop(0, NCHUNK // 2, _outer, 0)

    plsc.subcore_barrier()
    pltpu.sync_copy(accs.at[pl.ds(s * (NBKT // NSUB), NBKT // NSUB)],
                    outs_hbm.at[c, pl.ds(s * (NBKT // NSUB), NBKT // NSUB)])


def _edge_b_body(v_hbm, ea_hbm, src_hbm, dst_hbm, outwv_hbm,
                 srcv0, srcv1, dstv0, dstv1, vr0, vr1, eab0, eab1,
                 swv, zb, accwv, semv0, semv1, seme0, seme1):
    c = lax.axis_index("c")
    s = lax.axis_index("s")
    wid = c * NSUB + s
    srcv = (srcv0, srcv1)
    dstv = (dstv0, dstv1)
    vr = (vr0, vr1)
    eab = (eab0, eab1)
    semv = (semv0, semv1)
    seme = (seme0, seme1)

    zero16 = jnp.zeros((16,), _f32)
    iota16 = lax.iota(jnp.int32, 16)

    def _zzb(r, _):
        for j in range(HID // 16):
            zb[r, pl.ds(j * 16, 16)] = zero16
        return 0
    lax.fori_loop(0, 128, _zzb, 0)
    for t in range(5):
        pltpu.sync_copy(zb.at[pl.ds(0, 128)],
                        accwv.at[pl.ds(s * 624 + t * 128, 128)])
    plsc.subcore_barrier()

    def _issue(ci, b):
        ebase = wid * EPW + ci * CH
        pltpu.sync_copy(src_hbm.at[pl.ds(ebase, CH)], srcv[b])
        pltpu.sync_copy(dst_hbm.at[pl.ds(ebase, CH)], dstv[b])
        pltpu.async_copy(v_hbm.at[srcv[b]], vr[b], semv[b])
        pltpu.async_copy(ea_hbm.at[pl.ds(0, HEADS), pl.ds(ebase, CH)], eab[b], seme[b])

    _issue(0, 0)

    def _outer(i, _):
        for b in range(2):
            ci = 2 * i + b

            @pl.when(ci + 1 < NCHUNK)
            def _():
                _issue(ci + 1, 1 - b)

            ebase = wid * EPW + ci * CH
            pltpu.make_async_copy(v_hbm.at[srcv[b]], vr[b], semv[b]).wait()
            pltpu.make_async_copy(ea_hbm.at[pl.ds(0, HEADS), pl.ds(ebase, CH)],
                                  eab[b], seme[b]).wait()

            def _group(g, _, b=b):
                row_ids = g * 16 + iota16
                for h in range(HEADS):
                    ea_h = eab[b][h, pl.ds(g * 16, 16)]
                    for j in range(HC):
                        col = h * HC + ((iota16 + (j + g)) & 15)
                        vv = plsc.load_gather(vr[b], [row_ids, col])
                        plsc.store_scatter(swv, [row_ids, col], ea_h * vv)
                return 0
            lax.fori_loop(0, GRP, _group, 0)

            pltpu.sync_copy(swv, accwv.at[dstv[b]], add=True)
        return 0

    lax.fori_loop(0, NCHUNK // 2, _outer, 0)

    plsc.subcore_barrier()

    @pl.when(s < NSUB - 1)
    def _copy_main():
        pltpu.sync_copy(accwv.at[pl.ds(s * 624, 624)],
                        outwv_hbm.at[c, pl.ds(s * 624, 624)])

    @pl.when(s == NSUB - 1)
    def _copy_tail():
        pltpu.sync_copy(accwv.at[pl.ds((NSUB - 1) * 624, N - (NSUB - 1) * 624)],
                        outwv_hbm.at[c, pl.ds((NSUB - 1) * 624, N - (NSUB - 1) * 624)])


def _edge_phase(q, k, v, src, dst):
    ea, outs = pl.kernel(
        _edge_a_body,
        out_type=[jax.ShapeDtypeStruct((HEADS, E), _f32),
                  jax.ShapeDtypeStruct((NSC, NBKT, HID), _f32)],
        scratch_types=[
            pltpu.VMEM((CH,), jnp.int32),
            pltpu.VMEM((CH,), jnp.int32),
            pltpu.VMEM((CH,), jnp.int32),
            pltpu.VMEM((CH,), jnp.int32),
            pltpu.VMEM((CH,), jnp.int32),
            pltpu.VMEM((CH, HID), _f32),
            pltpu.VMEM((CH, HID), _f32),
            pltpu.VMEM((CH, HID), _f32),
            pltpu.VMEM((CH, HID), _f32),
            pltpu.VMEM((CH, HID), _f32),
            pltpu.VMEM((HEADS, CH), _f32),
            pltpu.VMEM((NBKT // NSUB, HID), _f32),
            pltpu.VMEM_SHARED((NBKT, HID), _f32),
            pltpu.SemaphoreType.DMA,
            pltpu.SemaphoreType.DMA,
            pltpu.SemaphoreType.DMA,
            pltpu.SemaphoreType.DMA,
        ],
        **_SC_MESH,
    )(q, k, src, dst)

    outwv = pl.kernel(
        _edge_b_body,
        out_type=jax.ShapeDtypeStruct((NSC, N, HID), _f32),
        scratch_types=[
            pltpu.VMEM((CH,), jnp.int32),
            pltpu.VMEM((CH,), jnp.int32),
            pltpu.VMEM((CH,), jnp.int32),
            pltpu.VMEM((CH,), jnp.int32),
            pltpu.VMEM((CH, HID), _f32),
            pltpu.VMEM((CH, HID), _f32),
            pltpu.VMEM((HEADS, CH), _f32),
            pltpu.VMEM((HEADS, CH), _f32),
            pltpu.VMEM((CH, HID), _f32),
            pltpu.VMEM((128, HID), _f32),
            pltpu.VMEM_SHARED((N, HID), _f32),
            pltpu.SemaphoreType.DMA,
            pltpu.SemaphoreType.DMA,
            pltpu.SemaphoreType.DMA,
            pltpu.SemaphoreType.DMA,
        ],
        **_SC_MESH,
    )(v, ea, src, dst)
    return outwv, outs


def _combine_body(s2_ref, wv2_ref, sh_ref, hr_ref, g_ref, b_ref, rep_ref,
                  o_ref, *, relu):
    svec = jnp.sum(s2_ref[...], axis=0)
    wv = jnp.sum(wv2_ref[...], axis=0)
    srep = jnp.dot(1.0 / (svec + 1e-16), rep_ref[...],
                   preferred_element_type=_f32)
    t = wv * srep + sh_ref[...] + hr_ref[...]
    mu = jnp.mean(t, axis=-1, keepdims=True)
    var = jnp.mean((t - mu) ** 2, axis=-1, keepdims=True)
    y = (t - mu) * lax.rsqrt(var + 1e-5) * g_ref[...] + b_ref[...]
    if relu:
        y = jnp.maximum(y, 0.0)
    o_ref[...] = y


def _combine(s2, wv2, sh, hres, g, b, rep, relu):
    bn = 2000
    body = functools.partial(_combine_body, relu=relu)
    return pl.pallas_call(
        body,
        grid=(N // bn,),
        in_specs=[
            pl.BlockSpec((NSC, bn, HEADS), lambda i: (0, i, 0)),
            pl.BlockSpec((NSC, bn, HID), lambda i: (0, i, 0)),
            pl.BlockSpec((bn, HID), lambda i: (i, 0)),
            pl.BlockSpec((bn, HID), lambda i: (i, 0)),
            pl.BlockSpec((1, HID), lambda i: (0, 0)),
            pl.BlockSpec((1, HID), lambda i: (0, 0)),
            pl.BlockSpec((HEADS, HID), lambda i: (0, 0)),
        ],
        out_specs=pl.BlockSpec((bn, HID), lambda i: (i, 0)),
        out_shape=jax.ShapeDtypeStruct((N, HID), _f32),
    )(s2, wv2, sh, hres, g, b, rep)


def _heads_body(h_ref, p1w_ref, p1b_ref, p2w_ref, p2b_ref, p3w_ref, p3b_ref,
                u1w_ref, u1b_ref, u2w_ref, u2b_ref, g1w_ref, g1b_ref,
                g2w_ref, g2b_ref, preds_o, unc_o, gc_o):
    h = h_ref[...]
    preds_cols = []
    unc_cols = []
    for d in range(D):
        a1 = jnp.maximum(jnp.dot(h, p1w_ref[d], preferred_element_type=_f32)
                         + p1b_ref[d][None, :], 0.0)
        a2 = jnp.maximum(jnp.dot(a1, p2w_ref[d], preferred_element_type=_f32)
                         + p2b_ref[d][None, :], 0.0)
        preds_cols.append(jnp.dot(a2, p3w_ref[d], preferred_element_type=_f32)
                          + p3b_ref[d][None, :])
        u1 = jnp.maximum(jnp.dot(h, u1w_ref[d], preferred_element_type=_f32)
                         + u1b_ref[d][None, :], 0.0)
        u2 = (jnp.dot(u1, u2w_ref[d], preferred_element_type=_f32)
              + u2b_ref[d][None, :])
        um = jnp.minimum(u2, 20.0)
        unc_cols.append(jnp.where(u2 > 20.0, u2,
                                  jnp.log(1.0 + jnp.exp(um))))
    preds_o[...] = jnp.concatenate(preds_cols, axis=1)
    unc_o[...] = jnp.concatenate(unc_cols, axis=1)
    gl = jnp.maximum(jnp.dot(h, g1w_ref[...], preferred_element_type=_f32)
                     + g1b_ref[...], 0.0)
    gl = jnp.dot(gl, g2w_ref[...], preferred_element_type=_f32) + g2b_ref[...]
    gc_o[...] = 1.0 / (1.0 + jnp.exp(-gl))


def _heads(h, P1W, P1b, P2W, P2b, P3W, P3b, U1W, U1b, U2W, U2b,
           G1W, G1b, G2W, G2b):
    bn = 1000
    full = lambda shape: pl.BlockSpec(shape, lambda i: tuple(0 for _ in shape))
    return pl.pallas_call(
        _heads_body,
        grid=(N // bn,),
        in_specs=[
            pl.BlockSpec((bn, HID), lambda i: (i, 0)),
            full((D, HID, HID // 2)), full((D, HID // 2)),
            full((D, HID // 2, HID // 4)), full((D, HID // 4)),
            full((D, HID // 4, 1)), full((D, 1)),
            full((D, HID, HID // 4)), full((D, HID // 4)),
            full((D, HID // 4, 1)), full((D, 1)),
            full((HID, HID // 2)), full((1, HID // 2)),
            full((HID // 2, 1)), full((1, 1)),
        ],
        out_specs=[pl.BlockSpec((bn, D), lambda i: (i, 0)),
                   pl.BlockSpec((bn, D), lambda i: (i, 0)),
                   pl.BlockSpec((bn, 1), lambda i: (i, 0))],
        out_shape=[jax.ShapeDtypeStruct((N, D), _f32),
                   jax.ShapeDtypeStruct((N, D), _f32),
                   jax.ShapeDtypeStruct((N, 1), _f32)],
    )(h, P1W, P1b, P2W, P2b, P3W, P3b, U1W, U1b, U2W, U2b,
      G1W, G1b[None, :], G2W, G2b[None, :])


def kernel(x, edge_index, missing_mask, Wf, bf, Emiss, Etype, Wp, bp,
           Wq, bq, Wk, bk, Wv, bv, Ws, bs, ln_g, ln_b,
           P1W, P1b, P2W, P2b, P3W, P3b, U1W, U1b, U2W, U2b,
           G1W, G1b, G2W, G2b):
    Wp0, Wp1, Wp2 = Wp[:Q], Wp[Q:2 * Q], Wp[2 * Q:]
    wfp = (Wf @ Wp0) / D
    u = (((Emiss[1] - Emiss[0]) @ Wp1) / D)[None, :]
    c0 = (bf.mean(0) @ Wp0 + Emiss[0] @ Wp1 + Etype[0] @ Wp2 + bp)[None, :]
    maskf = missing_mask.astype(_f32)

    h = _embed(x, maskf, wfp, u, c0)

    src = edge_index[0]
    dst = edge_index[1]
    rep = jnp.repeat(jnp.eye(HEADS, dtype=_f32), HC, axis=1)

    for l in range(NL):
        hres = h
        q, k, v, sh = _qkvs(h, Wq[l], bq[l][None, :], Wk[l], bk[l][None, :],
                            Wv[l], bv[l][None, :], Ws[l], bs[l][None, :])
        wv2, aggs = _edge_phase(q, k, v, src, dst)
        s2 = aggs.reshape(NSC, NBKT * HC, HEADS)[:, :N]
        h = _combine(s2, wv2, sh, hres, ln_g[l][None, :], ln_b[l][None, :],
                     rep, relu=(l < NL - 1))

    preds, unc, gc = _heads(h, P1W, P1b, P2W, P2b, P3W, P3b,
                            U1W, U1b, U2W, U2b, G1W, G1b, G2W, G2b)
    return (preds, unc, gc)


# speedup vs baseline: 77.0284x; 1.2827x over previous
import functools

import jax
import jax.numpy as jnp
from jax import lax
from jax.experimental import pallas as pl
from jax.experimental.pallas import tpu as pltpu
from jax.experimental.pallas import tpu_sc as plsc

N = 10000
D = 32
HID = 128
HEADS = 8
HC = 16
E = 640000
NL = 4
Q = HID // 4

NSC = 2
NSUB = 16
NW = NSC * NSUB
EPW = E // NW
CH = 80
NCHUNK = EPW // CH
GRP = CH // 16

_f32 = jnp.float32


def _embed_body(x_ref, mf_ref, wfp_ref, u_ref, c0_ref, o_ref):
    x = x_ref[...]
    cnt = jnp.sum(mf_ref[...], axis=1, keepdims=True)
    o_ref[...] = (jnp.dot(x, wfp_ref[...], preferred_element_type=_f32)
                  + cnt * u_ref[...] + c0_ref[...])


def _embed(x, maskf, wfp, u, c0):
    bn = 2000
    return pl.pallas_call(
        _embed_body,
        grid=(N // bn,),
        in_specs=[
            pl.BlockSpec((bn, D), lambda i: (i, 0)),
            pl.BlockSpec((bn, D), lambda i: (i, 0)),
            pl.BlockSpec((D, HID), lambda i: (0, 0)),
            pl.BlockSpec((1, HID), lambda i: (0, 0)),
            pl.BlockSpec((1, HID), lambda i: (0, 0)),
        ],
        out_specs=pl.BlockSpec((bn, HID), lambda i: (i, 0)),
        out_shape=jax.ShapeDtypeStruct((N, HID), _f32),
    )(x, maskf, wfp, u, c0)


def _qkvs_body(h_ref, wq_ref, bq_ref, wk_ref, bk_ref, wv_ref, bv_ref,
               ws_ref, bs_ref, q_o, k_o, v_o, s_o):
    h = h_ref[...]
    q_o[...] = jnp.dot(h, wq_ref[...], preferred_element_type=_f32) + bq_ref[...]
    k_o[...] = jnp.dot(h, wk_ref[...], preferred_element_type=_f32) + bk_ref[...]
    v_o[...] = jnp.dot(h, wv_ref[...], preferred_element_type=_f32) + bv_ref[...]
    s_o[...] = jnp.dot(h, ws_ref[...], preferred_element_type=_f32) + bs_ref[...]


def _qkvs(h, wq, bq, wk, bk, wv, bv, ws, bs):
    bn = 2000
    wspec = pl.BlockSpec((HID, HID), lambda i: (0, 0))
    bspec = pl.BlockSpec((1, HID), lambda i: (0, 0))
    ospec = pl.BlockSpec((bn, HID), lambda i: (i, 0))
    oshape = jax.ShapeDtypeStruct((N, HID), _f32)
    return pl.pallas_call(
        _qkvs_body,
        grid=(N // bn,),
        in_specs=[pl.BlockSpec((bn, HID), lambda i: (i, 0)),
                  wspec, bspec, wspec, bspec, wspec, bspec, wspec, bspec],
        out_specs=[ospec, ospec, ospec, ospec],
        out_shape=[oshape, oshape, oshape, oshape],
    )(h, wq, bq, wk, bk, wv, bv, ws, bs)


NBKT = 640


_SC_MESH = dict(
    mesh=plsc.VectorSubcoreMesh(core_axis_name="c", subcore_axis_name="s",
                                num_cores=NSC),
    compiler_params=pltpu.CompilerParams(needs_layout_passes=False,
                                         use_tc_tiling_on_sc=False),
)


def _edge_a_body(q_hbm, k_hbm, src_hbm, dst_hbm, ea_hbm, outs_hbm,
                 srcv0, srcv1, srcv2, srcv3, dstv0, dstv1, dstv2, dstv3,
                 bktv, qr0, qr1, kr0, kr1, ss, eab, zb, accs,
                 semi0, semi1, semi2, semi3, semq0, semq1, semk0, semk1):
    c = lax.axis_index("c")
    s = lax.axis_index("s")
    wid = c * NSUB + s
    srcv = (srcv0, srcv1, srcv2, srcv3)
    dstv = (dstv0, dstv1, dstv2, dstv3)
    qr = (qr0, qr1)
    kr = (kr0, kr1)
    semi = (semi0, semi1, semi2, semi3)
    semq = (semq0, semq1)
    semk = (semk0, semk1)

    zero16 = jnp.zeros((16,), _f32)
    iota16 = lax.iota(jnp.int32, 16)

    def _zss(r, _):
        for j in range(HID // 16):
            ss[r, pl.ds(j * 16, 16)] = zero16
        return 0
    lax.fori_loop(0, CH, _zss, 0)

    def _zzb(r, _):
        for j in range(HID // 16):
            zb[r, pl.ds(j * 16, 16)] = zero16
        return 0
    lax.fori_loop(0, NBKT // NSUB, _zzb, 0)
    pltpu.sync_copy(zb.at[pl.ds(0, NBKT // NSUB)],
                    accs.at[pl.ds(s * (NBKT // NSUB), NBKT // NSUB)])
    plsc.subcore_barrier()

    def _issue_idx(ci, p):
        ebase = wid * EPW + ci * CH
        pltpu.async_copy(src_hbm.at[pl.ds(ebase, CH)], srcv[p], semi[p])
        pltpu.async_copy(dst_hbm.at[pl.ds(ebase, CH)], dstv[p], semi[p])

    def _wait_idx(ci, p):
        ebase = wid * EPW + ci * CH
        pltpu.make_async_copy(src_hbm.at[pl.ds(ebase, CH)], srcv[p],
                              semi[p]).wait()
        pltpu.make_async_copy(dst_hbm.at[pl.ds(ebase, CH)], dstv[p],
                              semi[p]).wait()

    def _issue_gather(p, p2):
        pltpu.async_copy(q_hbm.at[dstv[p]], qr[p2], semq[p2])
        pltpu.async_copy(k_hbm.at[srcv[p]], kr[p2], semk[p2])

    _issue_idx(0, 0)
    _issue_idx(1, 1)
    _wait_idx(0, 0)
    _issue_gather(0, 0)

    def _outer(i, _):
        for p in range(4):
            ci = 4 * i + p
            p2 = p % 2

            @pl.when(ci + 2 < NCHUNK)
            def _():
                _issue_idx(ci + 2, (p + 2) % 4)

            @pl.when(ci + 1 < NCHUNK)
            def _():
                _wait_idx(ci + 1, (p + 1) % 4)
                _issue_gather((p + 1) % 4, (p + 1) % 2)

            @pl.when(ci < NCHUNK)
            def _():
                pltpu.make_async_copy(q_hbm.at[dstv[p]], qr[p2],
                                      semq[p2]).wait()
                pltpu.make_async_copy(k_hbm.at[srcv[p]], kr[p2],
                                      semk[p2]).wait()

                def _group(g, _, p=p, p2=p2):
                    row_ids = g * 16 + iota16
                    dv = dstv[p][pl.ds(g * 16, 16)]
                    lane0 = (dv & 15) * 8
                    bktv[pl.ds(g * 16, 16)] = lax.shift_right_logical(dv, 4)
                    for h in range(HEADS):
                        acc_v = zero16
                        for j in range(HC):
                            col = h * HC + ((iota16 + (j + g)) & 15)
                            qv = plsc.load_gather(qr[p2], [row_ids, col])
                            kv = plsc.load_gather(kr[p2], [row_ids, col])
                            acc_v = acc_v + qv * kv
                        ea = jnp.exp(acc_v * 0.25)
                        eab[h, pl.ds(g * 16, 16)] = ea
                        plsc.store_scatter(ss, [row_ids, lane0 + h], ea)
                    return 0
                lax.fori_loop(0, GRP, _group, 0)

                ebase = wid * EPW + ci * CH
                pltpu.sync_copy(ss, accs.at[bktv], add=True)
                pltpu.sync_copy(eab,
                                ea_hbm.at[pl.ds(0, HEADS), pl.ds(ebase, CH)])

                def _zgroup(g, _, p=p):
                    row_ids = g * 16 + iota16
                    dv = dstv[p][pl.ds(g * 16, 16)]
                    lane0 = (dv & 15) * 8
                    for h in range(HEADS):
                        plsc.store_scatter(ss, [row_ids, lane0 + h], zero16)
                    return 0
                lax.fori_loop(0, GRP, _zgroup, 0)
        return 0

    lax.fori_loop(0, (NCHUNK + 3) // 4, _outer, 0)

    plsc.subcore_barrier()
    pltpu.sync_copy(accs.at[pl.ds(s * (NBKT // NSUB), NBKT // NSUB)],
                    outs_hbm.at[c, pl.ds(s * (NBKT // NSUB), NBKT // NSUB)])


def _edge_b_body(v_hbm, ea_hbm, src_hbm, dst_hbm, outwv_hbm,
                 srcv0, srcv1, srcv2, srcv3, dstv0, dstv1, dstv2, dstv3,
                 vr0, vr1, eab0, eab1, swv, zb, accwv,
                 semi0, semi1, semi2, semi3, semv0, semv1, seme0, seme1):
    c = lax.axis_index("c")
    s = lax.axis_index("s")
    wid = c * NSUB + s
    srcv = (srcv0, srcv1, srcv2, srcv3)
    dstv = (dstv0, dstv1, dstv2, dstv3)
    vr = (vr0, vr1)
    eab = (eab0, eab1)
    semi = (semi0, semi1, semi2, semi3)
    semv = (semv0, semv1)
    seme = (seme0, seme1)

    zero16 = jnp.zeros((16,), _f32)
    iota16 = lax.iota(jnp.int32, 16)

    def _zzb(r, _):
        for j in range(HID // 16):
            zb[r, pl.ds(j * 16, 16)] = zero16
        return 0
    lax.fori_loop(0, 128, _zzb, 0)
    for t in range(5):
        pltpu.sync_copy(zb.at[pl.ds(0, 128)],
                        accwv.at[pl.ds(s * 624 + t * 128, 128)])
    plsc.subcore_barrier()

    def _issue_idx(ci, p):
        ebase = wid * EPW + ci * CH
        pltpu.async_copy(src_hbm.at[pl.ds(ebase, CH)], srcv[p], semi[p])
        pltpu.async_copy(dst_hbm.at[pl.ds(ebase, CH)], dstv[p], semi[p])

    def _wait_idx(ci, p):
        ebase = wid * EPW + ci * CH
        pltpu.make_async_copy(src_hbm.at[pl.ds(ebase, CH)], srcv[p],
                              semi[p]).wait()
        pltpu.make_async_copy(dst_hbm.at[pl.ds(ebase, CH)], dstv[p],
                              semi[p]).wait()

    def _issue_gather(ci, p, p2):
        ebase = wid * EPW + ci * CH
        pltpu.async_copy(v_hbm.at[srcv[p]], vr[p2], semv[p2])
        pltpu.async_copy(ea_hbm.at[pl.ds(0, HEADS), pl.ds(ebase, CH)],
                         eab[p2], seme[p2])

    _issue_idx(0, 0)
    _issue_idx(1, 1)
    _wait_idx(0, 0)
    _issue_gather(0, 0, 0)

    def _outer(i, _):
        for p in range(4):
            ci = 4 * i + p
            p2 = p % 2

            @pl.when(ci + 2 < NCHUNK)
            def _():
                _issue_idx(ci + 2, (p + 2) % 4)

            @pl.when(ci + 1 < NCHUNK)
            def _():
                _wait_idx(ci + 1, (p + 1) % 4)
                _issue_gather(ci + 1, (p + 1) % 4, (p + 1) % 2)

            @pl.when(ci < NCHUNK)
            def _():
                ebase = wid * EPW + ci * CH
                pltpu.make_async_copy(v_hbm.at[srcv[p]], vr[p2],
                                      semv[p2]).wait()
                pltpu.make_async_copy(
                    ea_hbm.at[pl.ds(0, HEADS), pl.ds(ebase, CH)], eab[p2],
                    seme[p2]).wait()

                def _group(g, _, p=p, p2=p2):
                    row_ids = g * 16 + iota16
                    for h in range(HEADS):
                        ea_h = eab[p2][h, pl.ds(g * 16, 16)]
                        for j in range(HC):
                            col = h * HC + ((iota16 + (j + g)) & 15)
                            vv = plsc.load_gather(vr[p2], [row_ids, col])
                            plsc.store_scatter(swv, [row_ids, col],
                                               ea_h * vv)
                    return 0
                lax.fori_loop(0, GRP, _group, 0)

                pltpu.sync_copy(swv, accwv.at[dstv[p]], add=True)
        return 0

    lax.fori_loop(0, (NCHUNK + 3) // 4, _outer, 0)

def _edge_phase(q, k, v, src, dst):
    ea, outs = pl.kernel(
        _edge_a_body,
        out_type=[jax.ShapeDtypeStruct((HEADS, E), _f32),
                  jax.ShapeDtypeStruct((NSC, NBKT, HID), _f32)],
        scratch_types=(
            [pltpu.VMEM((CH,), jnp.int32)] * 9
            + [pltpu.VMEM((CH, HID), _f32)] * 5
            + [pltpu.VMEM((HEADS, CH), _f32),
               pltpu.VMEM((NBKT // NSUB, HID), _f32),
               pltpu.VMEM_SHARED((NBKT, HID), _f32)]
            + [pltpu.SemaphoreType.DMA] * 8
        ),
        **_SC_MESH,
    )(q, k, src, dst)

    outwv = pl.kernel(
        _edge_b_body,
        out_type=jax.ShapeDtypeStruct((NSC, N, HID), _f32),
        scratch_types=(
            [pltpu.VMEM((CH,), jnp.int32)] * 8
            + [pltpu.VMEM((CH, HID), _f32)] * 2
            + [pltpu.VMEM((HEADS, CH), _f32)] * 2
            + [pltpu.VMEM((CH, HID), _f32),
               pltpu.VMEM((128, HID), _f32),
               pltpu.VMEM_SHARED((N, HID), _f32)]
            + [pltpu.SemaphoreType.DMA] * 8
        ),
        **_SC_MESH,
    )(v, ea, src, dst)
    return outwv, outs


def _combine_body(s2_ref, wv2_ref, sh_ref, hr_ref, g_ref, b_ref, rep_ref,
                  o_ref, *, relu):
    svec = jnp.sum(s2_ref[...], axis=0)
    wv = jnp.sum(wv2_ref[...], axis=0)
    srep = jnp.dot(1.0 / (svec + 1e-16), rep_ref[...],
                   preferred_element_type=_f32)
    t = wv * srep + sh_ref[...] + hr_ref[...]
    mu = jnp.mean(t, axis=-1, keepdims=True)
    var = jnp.mean((t - mu) ** 2, axis=-1, keepdims=True)
    y = (t - mu) * lax.rsqrt(var + 1e-5) * g_ref[...] + b_ref[...]
    if relu:
        y = jnp.maximum(y, 0.0)
    o_ref[...] = y


def _combine(s2, wv2, sh, hres, g, b, rep, relu):
    bn = 2000
    body = functools.partial(_combine_body, relu=relu)
    return pl.pallas_call(
        body,
        grid=(N // bn,),
        in_specs=[
            pl.BlockSpec((NSC, bn, HEADS), lambda i: (0, i, 0)),
            pl.BlockSpec((NSC, bn, HID), lambda i: (0, i, 0)),
            pl.BlockSpec((bn, HID), lambda i: (i, 0)),
            pl.BlockSpec((bn, HID), lambda i: (i, 0)),
            pl.BlockSpec((1, HID), lambda i: (0, 0)),
            pl.BlockSpec((1, HID), lambda i: (0, 0)),
            pl.BlockSpec((HEADS, HID), lambda i: (0, 0)),
        ],
        out_specs=pl.BlockSpec((bn, HID), lambda i: (i, 0)),
        out_shape=jax.ShapeDtypeStruct((N, HID), _f32),
    )(s2, wv2, sh, hres, g, b, rep)


def _heads_body(h_ref, p1w_ref, p1b_ref, p2w_ref, p2b_ref, p3w_ref, p3b_ref,
                u1w_ref, u1b_ref, u2w_ref, u2b_ref, g1w_ref, g1b_ref,
                g2w_ref, g2b_ref, preds_o, unc_o, gc_o):
    h = h_ref[...]
    preds_cols = []
    unc_cols = []
    for d in range(D):
        a1 = jnp.maximum(jnp.dot(h, p1w_ref[d], preferred_element_type=_f32)
                         + p1b_ref[d][None, :], 0.0)
        a2 = jnp.maximum(jnp.dot(a1, p2w_ref[d], preferred_element_type=_f32)
                         + p2b_ref[d][None, :], 0.0)
        preds_cols.append(jnp.dot(a2, p3w_ref[d], preferred_element_type=_f32)
                          + p3b_ref[d][None, :])
        u1 = jnp.maximum(jnp.dot(h, u1w_ref[d], preferred_element_type=_f32)
                         + u1b_ref[d][None, :], 0.0)
        u2 = (jnp.dot(u1, u2w_ref[d], preferred_element_type=_f32)
              + u2b_ref[d][None, :])
        um = jnp.minimum(u2, 20.0)
        unc_cols.append(jnp.where(u2 > 20.0, u2,
                                  jnp.log(1.0 + jnp.exp(um))))
    preds_o[...] = jnp.concatenate(preds_cols, axis=1)
    unc_o[...] = jnp.concatenate(unc_cols, axis=1)
    gl = jnp.maximum(jnp.dot(h, g1w_ref[...], preferred_element_type=_f32)
                     + g1b_ref[...], 0.0)
    gl = jnp.dot(gl, g2w_ref[...], preferred_element_type=_f32) + g2b_ref[...]
    gc_o[...] = 1.0 / (1.0 + jnp.exp(-gl))


def _heads(h, P1W, P1b, P2W, P2b, P3W, P3b, U1W, U1b, U2W, U2b,
           G1W, G1b, G2W, G2b):
    bn = 1000
    full = lambda shape: pl.BlockSpec(shape, lambda i: tuple(0 for _ in shape))
    return pl.pallas_call(
        _heads_body,
        grid=(N // bn,),
        in_specs=[
            pl.BlockSpec((bn, HID), lambda i: (i, 0)),
            full((D, HID, HID // 2)), full((D, HID // 2)),
            full((D, HID // 2, HID // 4)), full((D, HID // 4)),
            full((D, HID // 4, 1)), full((D, 1)),
            full((D, HID, HID // 4)), full((D, HID // 4)),
            full((D, HID // 4, 1)), full((D, 1)),
            full((HID, HID // 2)), full((1, HID // 2)),
            full((HID // 2, 1)), full((1, 1)),
        ],
        out_specs=[pl.BlockSpec((bn, D), lambda i: (i, 0)),
                   pl.BlockSpec((bn, D), lambda i: (i, 0)),
                   pl.BlockSpec((bn, 1), lambda i: (i, 0))],
        out_shape=[jax.ShapeDtypeStruct((N, D), _f32),
                   jax.ShapeDtypeStruct((N, D), _f32),
                   jax.ShapeDtypeStruct((N, 1), _f32)],
    )(h, P1W, P1b, P2W, P2b, P3W, P3b, U1W, U1b, U2W, U2b,
      G1W, G1b[None, :], G2W, G2b[None, :])


def kernel(x, edge_index, missing_mask, Wf, bf, Emiss, Etype, Wp, bp,
           Wq, bq, Wk, bk, Wv, bv, Ws, bs, ln_g, ln_b,
           P1W, P1b, P2W, P2b, P3W, P3b, U1W, U1b, U2W, U2b,
           G1W, G1b, G2W, G2b):
    Wp0, Wp1, Wp2 = Wp[:Q], Wp[Q:2 * Q], Wp[2 * Q:]
    wfp = (Wf @ Wp0) / D
    u = (((Emiss[1] - Emiss[0]) @ Wp1) / D)[None, :]
    c0 = (bf.mean(0) @ Wp0 + Emiss[0] @ Wp1 + Etype[0] @ Wp2 + bp)[None, :]
    maskf = missing_mask.astype(_f32)

    h = _embed(x, maskf, wfp, u, c0)

    src = edge_index[0]
    dst = edge_index[1]
    rep = jnp.repeat(jnp.eye(HEADS, dtype=_f32), HC, axis=1)

    for l in range(NL):
        hres = h
        q, k, v, sh = _qkvs(h, Wq[l], bq[l][None, :], Wk[l], bk[l][None, :],
                            Wv[l], bv[l][None, :], Ws[l], bs[l][None, :])
        wv2, aggs = _edge_phase(q, k, v, src, dst)
        s2 = aggs.reshape(NSC, NBKT * HC, HEADS)[:, :N]
        h = _combine(s2, wv2, sh, hres, ln_g[l][None, :], ln_b[l][None, :],
                     rep, relu=(l < NL - 1))

    preds, unc, gc = _heads(h, P1W, P1b, P2W, P2b, P3W, P3b,
                            U1W, U1b, U2W, U2b, G1W, G1b, G2W, G2b)
    return (preds, unc, gc)
